# baseline (speedup 1.0000x reference)
import functools

import jax
import jax.numpy as jnp
from jax import lax
from jax.experimental import pallas as pl
from jax.experimental.pallas import tpu as pltpu
from jax.experimental.pallas import tpu_sc as plsc

N = 10000
E = 320000
F = 128
H = 64
G = 64
NW = 32
CH = 128
NCH = 80
EPW = NCH * CH
EPAD = NW * EPW
AN = N + 16
NBUF = 4
RPT = 1000
BR = 2000
GRID = N // BR


def _sc_deg(dst3d):
    mesh = plsc.VectorSubcoreMesh(core_axis_name="c", subcore_axis_name="s")

    @functools.partial(
        pl.kernel, mesh=mesh,
        compiler_params=pltpu.CompilerParams(use_tc_tiling_on_sc=False),
        out_type=jax.ShapeDtypeStruct((2 * N, 16), jnp.float32),
        scratch_types=[
            pltpu.VMEM((NCH, CH), jnp.int32),
            pltpu.VMEM((CH, 16), jnp.float32),
            pltpu.VMEM((RPT, 16), jnp.float32),
            pltpu.VMEM_SHARED((AN, 16), jnp.float32),
        ],
    )
    def k(dst_hbm, out_hbm, idx_v, ones_v, z_v, acc):
        c = lax.axis_index("c")
        s = lax.axis_index("s")
        wid = s * 2 + c

        def fill_ones(i, carry):
            ones_v[i, :] = jnp.ones((16,), jnp.float32)
            return carry
        lax.fori_loop(0, CH, fill_ones, 0)

        pltpu.sync_copy(dst_hbm.at[wid], idx_v)

        @pl.when(s < N // RPT)
        def _init():
            def fill_zero(i, carry):
                z_v[i, :] = jnp.zeros((16,), jnp.float32)
                return carry
            lax.fori_loop(0, RPT, fill_zero, 0)
            pltpu.sync_copy(z_v, acc.at[pl.ds(s * RPT, RPT)])

        plsc.subcore_barrier()

        def body(j, carry):
            pltpu.sync_copy(ones_v, acc.at[idx_v.at[j]], add=True)
            return carry
        lax.fori_loop(0, NCH, body, 0)

        plsc.subcore_barrier()

        @pl.when(s < N // RPT)
        def _writeback():
            pltpu.sync_copy(acc.at[pl.ds(s * RPT, RPT)],
                            out_hbm.at[pl.ds(c * N + s * RPT, RPT)])

    return k(dst3d)


def _sc_agg(y, src3d, dst3d):
    mesh = plsc.VectorSubcoreMesh(core_axis_name="c", subcore_axis_name="s")

    @functools.partial(
        pl.kernel, mesh=mesh,
        compiler_params=pltpu.CompilerParams(use_tc_tiling_on_sc=False),
        out_type=jax.ShapeDtypeStruct((N, 2 * H), jnp.float32),
        scratch_types=[
            pltpu.VMEM((NCH, CH), jnp.int32),
            pltpu.VMEM((NCH, CH), jnp.int32),
            [pltpu.VMEM((CH, H), jnp.float32) for _ in range(NBUF)],
            pltpu.VMEM_SHARED((AN, H), jnp.float32),
            pltpu.SemaphoreType.DMA,
            pltpu.SemaphoreType.DMA,
        ],
    )
    def k(y_hbm, src_hbm, dst_hbm, out_hbm, src_v, dst_v, bufs, acc,
          gsem, ssem):
        c = lax.axis_index("c")
        s = lax.axis_index("s")
        wid = s * 2 + c
        pltpu.sync_copy(src_hbm.at[wid], src_v)
        pltpu.sync_copy(dst_hbm.at[wid], dst_v)

        @pl.when(s < N // RPT)
        def _init():
            pltpu.sync_copy(y_hbm.at[pl.ds(s * RPT, RPT)],
                            acc.at[pl.ds(s * RPT, RPT)])

        plsc.subcore_barrier()

        for b in range(NBUF):
            pltpu.async_copy(y_hbm.at[src_v.at[b]], bufs[b], gsem)

        def body(i, carry):
            j0 = i * NBUF
            for b in range(NBUF):
                pltpu.make_async_copy(
                    y_hbm.at[src_v.at[j0 + b]], bufs[b], gsem).wait()
                pltpu.async_copy(bufs[b], acc.at[dst_v.at[j0 + b]], ssem,
                                 add=True)
            for b in range(NBUF):
                @pl.when(j0 + b + NBUF < NCH)
                def _refill(b=b):
                    pltpu.make_async_copy(
                        bufs[b], acc.at[dst_v.at[j0 + b]], ssem).wait()
                    pltpu.async_copy(
                        y_hbm.at[src_v.at[j0 + b + NBUF]], bufs[b], gsem)
            return carry
        lax.fori_loop(0, NCH // NBUF, body, 0)

        for b in range(NBUF):
            pltpu.make_async_copy(
                bufs[b], acc.at[dst_v.at[NCH - NBUF + b]], ssem).wait()

        plsc.subcore_barrier()

        @pl.when(s < N // RPT)
        def _writeback():
            pltpu.sync_copy(acc.at[pl.ds(s * RPT, RPT)],
                            out_hbm.at[pl.ds(s * RPT, RPT), pl.ds(c * H, H)])

    return k(y, src3d, dst3d)


def _conv_mat(wrow, lin, lout):
    ii = lax.broadcasted_iota(jnp.int32, (lin, lout), 0)
    jj = lax.broadcasted_iota(jnp.int32, (lin, lout), 1)
    t = ii - 2 * jj
    acc = jnp.zeros((lin, lout), jnp.float32)
    for tap in range(10):
        acc = acc + jnp.where(t == tap, wrow[:, tap:tap + 1], 0.0)
    return acc


def _tc_pre(x, dp, conv_w, conv_b, g1w):
    def body(x_ref, dp0_ref, dp1_ref, cw_ref, cb_ref, g1w_ref,
             y_ref, dinv_ref, w_scr, c1_scr):
        i = pl.program_id(0)

        @pl.when(i == 0)
        def _build():
            wrow = cw_ref[...]
            c1m = _conv_mat(wrow, F, 60)
            c2m = _conv_mat(wrow, 60, 26)
            c3m = _conv_mat(wrow, 26, 9)
            m = jnp.dot(jnp.dot(c1m, c2m, preferred_element_type=jnp.float32),
                        c3m, preferred_element_type=jnp.float32)
            w_scr[...] = jnp.dot(m, g1w_ref[...],
                                 preferred_element_type=jnp.float32)
            ssum = jnp.sum(wrow, keepdims=True)
            beta = cb_ref[...] * (ssum * ssum + ssum + 1.0)
            c1_scr[...] = beta * jnp.sum(g1w_ref[...], axis=0, keepdims=True)

        deg = dp0_ref[:, 0:1] + dp1_ref[:, 0:1] + 1.0
        dinv = lax.rsqrt(deg)
        dinv_ref[...] = dinv
        t0 = (jnp.dot(x_ref[...], w_scr[...],
                      preferred_element_type=jnp.float32) + c1_scr[...])
        y_ref[...] = dinv * t0

    return pl.pallas_call(
        body,
        grid=(GRID,),
        in_specs=[
            pl.BlockSpec((BR, F), lambda i: (i, 0)),
            pl.BlockSpec((BR, 16), lambda i: (i, 0)),
            pl.BlockSpec((BR, 16), lambda i: (i + GRID, 0)),
            pl.BlockSpec((1, 10), lambda i: (0, 0)),
            pl.BlockSpec((1, 1), lambda i: (0, 0)),
            pl.BlockSpec((9, H), lambda i: (0, 0)),
        ],
        out_specs=[
            pl.BlockSpec((BR, H), lambda i: (i, 0)),
            pl.BlockSpec((BR, 1), lambda i: (i, 0)),
        ],
        out_shape=[
            jax.ShapeDtypeStruct((N, H), jnp.float32),
            jax.ShapeDtypeStruct((N, 1), jnp.float32),
        ],
        scratch_shapes=[
            pltpu.VMEM((F, H), jnp.float32),
            pltpu.VMEM((1, H), jnp.float32),
        ],
    )(x, dp, dp, conv_w, conv_b, g1w)


def _tc_mid(p, y_prev, dinv, bias, w):
    def body(p_ref, yp_ref, dinv_ref, b_ref, w_ref, y_ref):
        dinv = dinv_ref[...]
        sagg = p_ref[:, :H] + p_ref[:, H:] - yp_ref[...]
        h = jnp.maximum(dinv * sagg + b_ref[...], 0.0)
        y_ref[...] = dinv * jnp.dot(h, w_ref[...],
                                    preferred_element_type=jnp.float32)

    return pl.pallas_call(
        body,
        grid=(GRID,),
        in_specs=[
            pl.BlockSpec((BR, 2 * H), lambda i: (i, 0)),
            pl.BlockSpec((BR, H), lambda i: (i, 0)),
            pl.BlockSpec((BR, 1), lambda i: (i, 0)),
            pl.BlockSpec((1, H), lambda i: (0, 0)),
            pl.BlockSpec((H, H), lambda i: (0, 0)),
        ],
        out_specs=pl.BlockSpec((BR, H), lambda i: (i, 0)),
        out_shape=jax.ShapeDtypeStruct((N, H), jnp.float32),
    )(p, y_prev, dinv, bias, w)


def _tc_post(p, y3, dinv, g3b, batch2d, l1w, l1b, l2w, l2b):
    def body(p_ref, y3_ref, dinv_ref, b_ref, bat_ref,
             l1w_ref, l1b_ref, l2w_ref, l2b_ref, out_ref,
             pooled_scr, cnt_scr):
        i = pl.program_id(0)
        h = (dinv_ref[...] * (p_ref[:, :H] + p_ref[:, H:] - y3_ref[...])
             + b_ref[...])
        gids = lax.broadcasted_iota(jnp.int32, (1, G), 1)
        onehot = (bat_ref[...] == gids).astype(jnp.float32)
        dn = (((0,), (0,)), ((), ()))
        pooled_inc = lax.dot_general(onehot, h, dn,
                                     preferred_element_type=jnp.float32)
        cnt_inc = lax.dot_general(onehot, jnp.ones((BR, 1), jnp.float32), dn,
                                  preferred_element_type=jnp.float32)

        @pl.when(i == 0)
        def _init():
            pooled_scr[...] = pooled_inc
            cnt_scr[...] = cnt_inc

        @pl.when(i > 0)
        def _acc():
            pooled_scr[...] = pooled_scr[...] + pooled_inc
            cnt_scr[...] = cnt_scr[...] + cnt_inc

        @pl.when(i == GRID - 1)
        def _final():
            pooled = pooled_scr[...] / jnp.maximum(cnt_scr[...], 1.0)
            dn1 = (((1,), (1,)), ((), ()))
            z = jnp.maximum(
                lax.dot_general(pooled, l1w_ref[...], dn1,
                                preferred_element_type=jnp.float32)
                + l1b_ref[...], 0.0)
            o = (jnp.sum(z * l2w_ref[...], axis=1, keepdims=True)
                 + l2b_ref[0, 0])
            out_ref[...] = jax.nn.sigmoid(o)

    return pl.pallas_call(
        body,
        grid=(GRID,),
        in_specs=[
            pl.BlockSpec((BR, 2 * H), lambda i: (i, 0)),
            pl.BlockSpec((BR, H), lambda i: (i, 0)),
            pl.BlockSpec((BR, 1), lambda i: (i, 0)),
            pl.BlockSpec((1, H), lambda i: (0, 0)),
            pl.BlockSpec((BR, 1), lambda i: (i, 0)),
            pl.BlockSpec((32, H), lambda i: (0, 0)),
            pl.BlockSpec((1, 32), lambda i: (0, 0)),
            pl.BlockSpec((1, 32), lambda i: (0, 0)),
            pl.BlockSpec((1, 1), lambda i: (0, 0)),
        ],
        out_specs=pl.BlockSpec((G, 1), lambda i: (0, 0)),
        out_shape=jax.ShapeDtypeStruct((G, 1), jnp.float32),
        scratch_shapes=[
            pltpu.VMEM((G, G), jnp.float32),
            pltpu.VMEM((G, 1), jnp.float32),
        ],
    )(p, y3, dinv, g3b, batch2d, l1w, l1b, l2w, l2b)


def kernel(x, edge_index, batch, conv_w, conv_b, g1w, g1b, g2w, g2b,
           g3w, g3b, l1w, l1b, l2w, l2b):
    pad = EPAD - E
    src3d = jnp.concatenate(
        [edge_index[0], jnp.zeros((pad,), jnp.int32)]).reshape(NW, NCH, CH)
    dst3d = jnp.concatenate(
        [edge_index[1], jnp.full((pad,), N, jnp.int32)]).reshape(NW, NCH, CH)
    cw = conv_w.reshape(1, 10)
    cb = conv_b.reshape(1, 1)

    dp = _sc_deg(dst3d)
    y1, dinv = _tc_pre(x, dp, cw, cb, g1w)
    p1 = _sc_agg(y1, src3d, dst3d)
    y2 = _tc_mid(p1, y1, dinv, g1b.reshape(1, H), g2w)
    p2 = _sc_agg(y2, src3d, dst3d)
    y3 = _tc_mid(p2, y2, dinv, g2b.reshape(1, H), g3w)
    p3 = _sc_agg(y3, src3d, dst3d)
    out = _tc_post(p3, y3, dinv, g3b.reshape(1, H),
                   batch.reshape(N, 1), l1w, l1b.reshape(1, 32),
                   l2w, l2b.reshape(1, 1))
    return out


# speedup vs baseline: 1.0025x; 1.0025x over previous
import functools

import jax
import jax.numpy as jnp
from jax import lax
from jax.experimental import pallas as pl
from jax.experimental.pallas import tpu as pltpu
from jax.experimental.pallas import tpu_sc as plsc

N = 10000
E = 320000
F = 128
H = 64
G = 64
NW = 32
CH = 128
NCH = 80
EPW = NCH * CH
EPAD = NW * EPW
PADROWS = 2048
AN = N + PADROWS
NBUF = 4
RPT = 1000
BR = 2000
GRID = N // BR


def _sc_deg(dst3d):
    mesh = plsc.VectorSubcoreMesh(core_axis_name="c", subcore_axis_name="s")

    @functools.partial(
        pl.kernel, mesh=mesh,
        compiler_params=pltpu.CompilerParams(use_tc_tiling_on_sc=False),
        out_type=jax.ShapeDtypeStruct((2 * N, 16), jnp.float32),
        scratch_types=[
            pltpu.VMEM((NCH, CH), jnp.int32),
            pltpu.VMEM((CH, 16), jnp.float32),
            pltpu.VMEM((RPT, 16), jnp.float32),
            pltpu.VMEM_SHARED((AN, 16), jnp.float32),
        ],
    )
    def k(dst_hbm, out_hbm, idx_v, ones_v, z_v, acc):
        c = lax.axis_index("c")
        s = lax.axis_index("s")
        wid = s * 2 + c

        def fill_ones(i, carry):
            ones_v[i, :] = jnp.ones((16,), jnp.float32)
            return carry
        lax.fori_loop(0, CH, fill_ones, 0)

        pltpu.sync_copy(dst_hbm.at[wid], idx_v)

        @pl.when(s < N // RPT)
        def _init():
            def fill_zero(i, carry):
                z_v[i, :] = jnp.zeros((16,), jnp.float32)
                return carry
            lax.fori_loop(0, RPT, fill_zero, 0)
            pltpu.sync_copy(z_v, acc.at[pl.ds(s * RPT, RPT)])

        plsc.subcore_barrier()

        def body(j, carry):
            pltpu.sync_copy(ones_v, acc.at[idx_v.at[j]], add=True)
            return carry
        lax.fori_loop(0, NCH, body, 0)

        plsc.subcore_barrier()

        @pl.when(s < N // RPT)
        def _writeback():
            pltpu.sync_copy(acc.at[pl.ds(s * RPT, RPT)],
                            out_hbm.at[pl.ds(c * N + s * RPT, RPT)])

    return k(dst3d)


def _sc_agg(y, src3d, dst3d):
    mesh = plsc.VectorSubcoreMesh(core_axis_name="c", subcore_axis_name="s")

    @functools.partial(
        pl.kernel, mesh=mesh,
        compiler_params=pltpu.CompilerParams(use_tc_tiling_on_sc=False),
        out_type=jax.ShapeDtypeStruct((N, 2 * H), jnp.float32),
        scratch_types=[
            pltpu.VMEM((NCH, CH), jnp.int32),
            pltpu.VMEM((NCH, CH), jnp.int32),
            [pltpu.VMEM((CH, H), jnp.float32) for _ in range(NBUF)],
            pltpu.VMEM_SHARED((AN, H), jnp.float32),
            pltpu.SemaphoreType.DMA,
            pltpu.SemaphoreType.DMA,
        ],
    )
    def k(y_hbm, src_hbm, dst_hbm, out_hbm, src_v, dst_v, bufs, acc,
          gsem, ssem):
        c = lax.axis_index("c")
        s = lax.axis_index("s")
        wid = s * 2 + c
        pltpu.sync_copy(src_hbm.at[wid], src_v)
        pltpu.sync_copy(dst_hbm.at[wid], dst_v)

        @pl.when(s < N // RPT)
        def _init():
            pltpu.sync_copy(y_hbm.at[pl.ds(s * RPT, RPT)],
                            acc.at[pl.ds(s * RPT, RPT)])

        plsc.subcore_barrier()

        for b in range(NBUF):
            pltpu.async_copy(y_hbm.at[src_v.at[b]], bufs[b], gsem)

        def body(i, carry):
            j0 = i * NBUF
            for b in range(NBUF):
                pltpu.make_async_copy(
                    y_hbm.at[src_v.at[j0 + b]], bufs[b], gsem).wait()
                pltpu.async_copy(bufs[b], acc.at[dst_v.at[j0 + b]], ssem,
                                 add=True)
            for b in range(NBUF):
                @pl.when(j0 + b + NBUF < NCH)
                def _refill(b=b):
                    pltpu.make_async_copy(
                        bufs[b], acc.at[dst_v.at[j0 + b]], ssem).wait()
                    pltpu.async_copy(
                        y_hbm.at[src_v.at[j0 + b + NBUF]], bufs[b], gsem)
            return carry
        lax.fori_loop(0, NCH // NBUF, body, 0)

        for b in range(NBUF):
            pltpu.make_async_copy(
                bufs[b], acc.at[dst_v.at[NCH - NBUF + b]], ssem).wait()

        plsc.subcore_barrier()

        @pl.when(s < N // RPT)
        def _writeback():
            pltpu.sync_copy(acc.at[pl.ds(s * RPT, RPT)],
                            out_hbm.at[pl.ds(s * RPT, RPT), pl.ds(c * H, H)])

    return k(y, src3d, dst3d)


def _conv_mat(wrow, lin, lout):
    ii = lax.broadcasted_iota(jnp.int32, (lin, lout), 0)
    jj = lax.broadcasted_iota(jnp.int32, (lin, lout), 1)
    t = ii - 2 * jj
    acc = jnp.zeros((lin, lout), jnp.float32)
    for tap in range(10):
        acc = acc + jnp.where(t == tap, wrow[:, tap:tap + 1], 0.0)
    return acc


def _tc_pre(x, dp, conv_w, conv_b, g1w):
    def body(x_ref, dp0_ref, dp1_ref, cw_ref, cb_ref, g1w_ref,
             y_ref, dinv_ref, w_scr, c1_scr):
        i = pl.program_id(0)

        @pl.when(i == 0)
        def _build():
            wrow = cw_ref[...]
            c1m = _conv_mat(wrow, F, 60)
            c2m = _conv_mat(wrow, 60, 26)
            c3m = _conv_mat(wrow, 26, 9)
            m = jnp.dot(jnp.dot(c1m, c2m, preferred_element_type=jnp.float32),
                        c3m, preferred_element_type=jnp.float32)
            w_scr[...] = jnp.dot(m, g1w_ref[...],
                                 preferred_element_type=jnp.float32)
            ssum = jnp.sum(wrow, keepdims=True)
            beta = cb_ref[...] * (ssum * ssum + ssum + 1.0)
            c1_scr[...] = beta * jnp.sum(g1w_ref[...], axis=0, keepdims=True)

        deg = dp0_ref[:, 0:1] + dp1_ref[:, 0:1] + 1.0
        dinv = lax.rsqrt(deg)
        dinv_ref[...] = dinv
        t0 = (jnp.dot(x_ref[...], w_scr[...],
                      preferred_element_type=jnp.float32) + c1_scr[...])
        y_ref[...] = dinv * t0

    return pl.pallas_call(
        body,
        grid=(GRID,),
        in_specs=[
            pl.BlockSpec((BR, F), lambda i: (i, 0)),
            pl.BlockSpec((BR, 16), lambda i: (i, 0)),
            pl.BlockSpec((BR, 16), lambda i: (i + GRID, 0)),
            pl.BlockSpec((1, 10), lambda i: (0, 0)),
            pl.BlockSpec((1, 1), lambda i: (0, 0)),
            pl.BlockSpec((9, H), lambda i: (0, 0)),
        ],
        out_specs=[
            pl.BlockSpec((BR, H), lambda i: (i, 0)),
            pl.BlockSpec((BR, 1), lambda i: (i, 0)),
        ],
        out_shape=[
            jax.ShapeDtypeStruct((N, H), jnp.float32),
            jax.ShapeDtypeStruct((N, 1), jnp.float32),
        ],
        scratch_shapes=[
            pltpu.VMEM((F, H), jnp.float32),
            pltpu.VMEM((1, H), jnp.float32),
        ],
    )(x, dp, dp, conv_w, conv_b, g1w)


def _tc_mid(p, y_prev, dinv, bias, w):
    def body(p_ref, yp_ref, dinv_ref, b_ref, w_ref, y_ref):
        dinv = dinv_ref[...]
        sagg = p_ref[:, :H] + p_ref[:, H:] - yp_ref[...]
        h = jnp.maximum(dinv * sagg + b_ref[...], 0.0)
        y_ref[...] = dinv * jnp.dot(h, w_ref[...],
                                    preferred_element_type=jnp.float32)

    return pl.pallas_call(
        body,
        grid=(GRID,),
        in_specs=[
            pl.BlockSpec((BR, 2 * H), lambda i: (i, 0)),
            pl.BlockSpec((BR, H), lambda i: (i, 0)),
            pl.BlockSpec((BR, 1), lambda i: (i, 0)),
            pl.BlockSpec((1, H), lambda i: (0, 0)),
            pl.BlockSpec((H, H), lambda i: (0, 0)),
        ],
        out_specs=pl.BlockSpec((BR, H), lambda i: (i, 0)),
        out_shape=jax.ShapeDtypeStruct((N, H), jnp.float32),
    )(p, y_prev, dinv, bias, w)


def _tc_post(p, y3, dinv, g3b, batch2d, l1w, l1b, l2w, l2b):
    def body(p_ref, y3_ref, dinv_ref, b_ref, bat_ref,
             l1w_ref, l1b_ref, l2w_ref, l2b_ref, out_ref,
             pooled_scr, cnt_scr):
        i = pl.program_id(0)
        h = (dinv_ref[...] * (p_ref[:, :H] + p_ref[:, H:] - y3_ref[...])
             + b_ref[...])
        gids = lax.broadcasted_iota(jnp.int32, (1, G), 1)
        onehot = (bat_ref[...] == gids).astype(jnp.float32)
        dn = (((0,), (0,)), ((), ()))
        pooled_inc = lax.dot_general(onehot, h, dn,
                                     preferred_element_type=jnp.float32)
        cnt_inc = lax.dot_general(onehot, jnp.ones((BR, 1), jnp.float32), dn,
                                  preferred_element_type=jnp.float32)

        @pl.when(i == 0)
        def _init():
            pooled_scr[...] = pooled_inc
            cnt_scr[...] = cnt_inc

        @pl.when(i > 0)
        def _acc():
            pooled_scr[...] = pooled_scr[...] + pooled_inc
            cnt_scr[...] = cnt_scr[...] + cnt_inc

        @pl.when(i == GRID - 1)
        def _final():
            pooled = pooled_scr[...] / jnp.maximum(cnt_scr[...], 1.0)
            dn1 = (((1,), (1,)), ((), ()))
            z = jnp.maximum(
                lax.dot_general(pooled, l1w_ref[...], dn1,
                                preferred_element_type=jnp.float32)
                + l1b_ref[...], 0.0)
            o = (jnp.sum(z * l2w_ref[...], axis=1, keepdims=True)
                 + l2b_ref[0, 0])
            out_ref[...] = jax.nn.sigmoid(o)

    return pl.pallas_call(
        body,
        grid=(GRID,),
        in_specs=[
            pl.BlockSpec((BR, 2 * H), lambda i: (i, 0)),
            pl.BlockSpec((BR, H), lambda i: (i, 0)),
            pl.BlockSpec((BR, 1), lambda i: (i, 0)),
            pl.BlockSpec((1, H), lambda i: (0, 0)),
            pl.BlockSpec((BR, 1), lambda i: (i, 0)),
            pl.BlockSpec((32, H), lambda i: (0, 0)),
            pl.BlockSpec((1, 32), lambda i: (0, 0)),
            pl.BlockSpec((1, 32), lambda i: (0, 0)),
            pl.BlockSpec((1, 1), lambda i: (0, 0)),
        ],
        out_specs=pl.BlockSpec((G, 1), lambda i: (0, 0)),
        out_shape=jax.ShapeDtypeStruct((G, 1), jnp.float32),
        scratch_shapes=[
            pltpu.VMEM((G, G), jnp.float32),
            pltpu.VMEM((G, 1), jnp.float32),
        ],
    )(p, y3, dinv, g3b, batch2d, l1w, l1b, l2w, l2b)


def kernel(x, edge_index, batch, conv_w, conv_b, g1w, g1b, g2w, g2b,
           g3w, g3b, l1w, l1b, l2w, l2b):
    pad = EPAD - E
    src3d = jnp.concatenate(
        [edge_index[0], jnp.zeros((pad,), jnp.int32)]).reshape(NW, NCH, CH)
    dst3d = jnp.concatenate(
        [edge_index[1],
         N + (jnp.arange(pad, dtype=jnp.int32) % PADROWS)]
    ).reshape(NW, NCH, CH)
    cw = conv_w.reshape(1, 10)
    cb = conv_b.reshape(1, 1)

    dp = _sc_deg(dst3d)
    y1, dinv = _tc_pre(x, dp, cw, cb, g1w)
    p1 = _sc_agg(y1, src3d, dst3d)
    y2 = _tc_mid(p1, y1, dinv, g1b.reshape(1, H), g2w)
    p2 = _sc_agg(y2, src3d, dst3d)
    y3 = _tc_mid(p2, y2, dinv, g2b.reshape(1, H), g3w)
    p3 = _sc_agg(y3, src3d, dst3d)
    out = _tc_post(p3, y3, dinv, g3b.reshape(1, H),
                   batch.reshape(N, 1), l1w, l1b.reshape(1, 32),
                   l2w, l2b.reshape(1, 1))
    return out


# speedup vs baseline: 2.8877x; 2.8804x over previous
import functools

import jax
import jax.numpy as jnp
from jax import lax
from jax.experimental import pallas as pl
from jax.experimental.pallas import tpu as pltpu
from jax.experimental.pallas import tpu_sc as plsc

N = 10000
E = 320000
F = 128
H = 64
G = 64
NW = 32
CH = 128
NCH = 80
EPW = NCH * CH
EPAD = NW * EPW
PADROWS = 2048
AN = N + PADROWS
NBUF = 4
RPT = 1000
BR = 2000
GRID = N // BR


def _sc_deg(dst3d):
    mesh = plsc.VectorSubcoreMesh(core_axis_name="c", subcore_axis_name="s")

    @functools.partial(
        pl.kernel, mesh=mesh,
        compiler_params=pltpu.CompilerParams(use_tc_tiling_on_sc=False),
        out_type=jax.ShapeDtypeStruct((2 * N, 16), jnp.float32),
        scratch_types=[
            pltpu.VMEM((NCH, CH), jnp.int32),
            pltpu.VMEM((CH, 16), jnp.float32),
            pltpu.VMEM((RPT, 16), jnp.float32),
            pltpu.VMEM_SHARED((AN, 16), jnp.float32),
        ],
    )
    def k(dst_hbm, out_hbm, idx_v, ones_v, z_v, acc):
        c = lax.axis_index("c")
        s = lax.axis_index("s")
        wid = s * 2 + c

        def fill_ones(i, carry):
            ones_v[i, :] = jnp.ones((16,), jnp.float32)
            return carry
        lax.fori_loop(0, CH, fill_ones, 0)

        pltpu.sync_copy(dst_hbm.at[wid], idx_v)

        @pl.when(s < N // RPT)
        def _init():
            def fill_zero(i, carry):
                z_v[i, :] = jnp.zeros((16,), jnp.float32)
                return carry
            lax.fori_loop(0, RPT, fill_zero, 0)
            pltpu.sync_copy(z_v, acc.at[pl.ds(s * RPT, RPT)])

        plsc.subcore_barrier()

        def body(j, carry):
            pltpu.sync_copy(ones_v, acc.at[idx_v.at[j]], add=True)
            return carry
        lax.fori_loop(0, NCH, body, 0)

        plsc.subcore_barrier()

        @pl.when(s < N // RPT)
        def _writeback():
            pltpu.sync_copy(acc.at[pl.ds(s * RPT, RPT)],
                            out_hbm.at[pl.ds(c * N + s * RPT, RPT)])

    return k(dst3d)


def _sc_agg(y, src3d, dst3d):
    mesh = plsc.VectorSubcoreMesh(core_axis_name="c", subcore_axis_name="s")

    @functools.partial(
        pl.kernel, mesh=mesh,
        compiler_params=pltpu.CompilerParams(use_tc_tiling_on_sc=False),
        out_type=jax.ShapeDtypeStruct((N, 2 * H), jnp.float32),
        scratch_types=[
            pltpu.VMEM((NCH, CH), jnp.int32),
            pltpu.VMEM((NCH, CH), jnp.int32),
            [pltpu.VMEM((CH, H), jnp.float32) for _ in range(NBUF)],
            pltpu.VMEM_SHARED((AN, H), jnp.float32),
            pltpu.SemaphoreType.DMA,
            pltpu.SemaphoreType.DMA,
        ],
    )
    def k(y_hbm, src_hbm, dst_hbm, out_hbm, src_v, dst_v, bufs, acc,
          gsem, ssem):
        c = lax.axis_index("c")
        s = lax.axis_index("s")
        wid = s * 2 + c
        pltpu.sync_copy(src_hbm.at[wid], src_v)
        pltpu.sync_copy(dst_hbm.at[wid], dst_v)

        @pl.when(s < N // RPT)
        def _init():
            pltpu.sync_copy(y_hbm.at[pl.ds(s * RPT, RPT)],
                            acc.at[pl.ds(s * RPT, RPT)])

        plsc.subcore_barrier()

        for b in range(NBUF):
            pltpu.async_copy(y_hbm.at[src_v.at[b]], bufs[b], gsem)

        def body(i, carry):
            j0 = i * NBUF
            for b in range(NBUF):
                pltpu.make_async_copy(
                    y_hbm.at[src_v.at[j0 + b]], bufs[b], gsem).wait()
                pltpu.async_copy(bufs[b], acc.at[dst_v.at[j0 + b]], ssem,
                                 add=True)
            for b in range(NBUF):
                @pl.when(j0 + b + NBUF < NCH)
                def _refill(b=b):
                    pltpu.make_async_copy(
                        bufs[b], acc.at[dst_v.at[j0 + b]], ssem).wait()
                    pltpu.async_copy(
                        y_hbm.at[src_v.at[j0 + b + NBUF]], bufs[b], gsem)
            return carry
        lax.fori_loop(0, NCH // NBUF, body, 0)

        for b in range(NBUF):
            pltpu.make_async_copy(
                bufs[b], acc.at[dst_v.at[NCH - NBUF + b]], ssem).wait()

        plsc.subcore_barrier()

        @pl.when(s < N // RPT)
        def _writeback():
            pltpu.sync_copy(acc.at[pl.ds(s * RPT, RPT)],
                            out_hbm.at[pl.ds(s * RPT, RPT), pl.ds(c * H, H)])

    return k(y, src3d, dst3d)


def _conv_mat(wrow, lin, lout):
    ii = lax.broadcasted_iota(jnp.int32, (lin, lout), 0)
    jj = lax.broadcasted_iota(jnp.int32, (lin, lout), 1)
    t = ii - 2 * jj
    acc = jnp.zeros((lin, lout), jnp.float32)
    for tap in range(10):
        acc = acc + jnp.where(t == tap, wrow[:, tap:tap + 1], 0.0)
    return acc


def _tc_pre(x, dp, conv_w, conv_b, g1w):
    def body(x_ref, dp0_ref, dp1_ref, cw_ref, cb_ref, g1w_ref,
             y_ref, dinv_ref, w_scr, c1_scr):
        i = pl.program_id(0)

        @pl.when(i == 0)
        def _build():
            wrow = cw_ref[...]
            c1m = _conv_mat(wrow, F, 60)
            c2m = _conv_mat(wrow, 60, 26)
            c3m = _conv_mat(wrow, 26, 9)
            m = jnp.dot(jnp.dot(c1m, c2m, preferred_element_type=jnp.float32),
                        c3m, preferred_element_type=jnp.float32)
            w_scr[...] = jnp.dot(m, g1w_ref[...],
                                 preferred_element_type=jnp.float32)
            ssum = jnp.sum(wrow, keepdims=True)
            beta = cb_ref[...] * (ssum * ssum + ssum + 1.0)
            c1_scr[...] = beta * jnp.sum(g1w_ref[...], axis=0, keepdims=True)

        deg = dp0_ref[:, 0:1] + dp1_ref[:, 0:1] + 1.0
        dinv = lax.rsqrt(deg)
        dinv_ref[...] = dinv
        t0 = (jnp.dot(x_ref[...], w_scr[...],
                      preferred_element_type=jnp.float32) + c1_scr[...])
        y_ref[...] = dinv * t0

    return pl.pallas_call(
        body,
        grid=(GRID,),
        in_specs=[
            pl.BlockSpec((BR, F), lambda i: (i, 0)),
            pl.BlockSpec((BR, 16), lambda i: (i, 0)),
            pl.BlockSpec((BR, 16), lambda i: (i + GRID, 0)),
            pl.BlockSpec((1, 10), lambda i: (0, 0)),
            pl.BlockSpec((1, 1), lambda i: (0, 0)),
            pl.BlockSpec((9, H), lambda i: (0, 0)),
        ],
        out_specs=[
            pl.BlockSpec((BR, H), lambda i: (i, 0)),
            pl.BlockSpec((BR, 1), lambda i: (i, 0)),
        ],
        out_shape=[
            jax.ShapeDtypeStruct((N, H), jnp.float32),
            jax.ShapeDtypeStruct((N, 1), jnp.float32),
        ],
        scratch_shapes=[
            pltpu.VMEM((F, H), jnp.float32),
            pltpu.VMEM((1, H), jnp.float32),
        ],
    )(x, dp, dp, conv_w, conv_b, g1w)


def _tc_mid(p, y_prev, dinv, bias, w):
    def body(p_ref, yp_ref, dinv_ref, b_ref, w_ref, y_ref):
        dinv = dinv_ref[...]
        sagg = p_ref[:, :H] + p_ref[:, H:] - yp_ref[...]
        h = jnp.maximum(dinv * sagg + b_ref[...], 0.0)
        y_ref[...] = dinv * jnp.dot(h, w_ref[...],
                                    preferred_element_type=jnp.float32)

    return pl.pallas_call(
        body,
        grid=(GRID,),
        in_specs=[
            pl.BlockSpec((BR, 2 * H), lambda i: (i, 0)),
            pl.BlockSpec((BR, H), lambda i: (i, 0)),
            pl.BlockSpec((BR, 1), lambda i: (i, 0)),
            pl.BlockSpec((1, H), lambda i: (0, 0)),
            pl.BlockSpec((H, H), lambda i: (0, 0)),
        ],
        out_specs=pl.BlockSpec((BR, H), lambda i: (i, 0)),
        out_shape=jax.ShapeDtypeStruct((N, H), jnp.float32),
    )(p, y_prev, dinv, bias, w)


def _tc_post(p, y3, dinv, g3b, batch2d, l1w, l1b, l2w, l2b):
    def body(p_ref, y3_ref, dinv_ref, b_ref, bat_ref,
             l1w_ref, l1b_ref, l2w_ref, l2b_ref, out_ref,
             pooled_scr, cnt_scr):
        i = pl.program_id(0)
        h = (dinv_ref[...] * (p_ref[:, :H] + p_ref[:, H:] - y3_ref[...])
             + b_ref[...])
        gids = lax.broadcasted_iota(jnp.int32, (1, G), 1)
        onehot = (bat_ref[...] == gids).astype(jnp.float32)
        dn = (((0,), (0,)), ((), ()))
        pooled_inc = lax.dot_general(onehot, h, dn,
                                     preferred_element_type=jnp.float32)
        cnt_inc = lax.dot_general(onehot, jnp.ones((BR, 1), jnp.float32), dn,
                                  preferred_element_type=jnp.float32)

        @pl.when(i == 0)
        def _init():
            pooled_scr[...] = pooled_inc
            cnt_scr[...] = cnt_inc

        @pl.when(i > 0)
        def _acc():
            pooled_scr[...] = pooled_scr[...] + pooled_inc
            cnt_scr[...] = cnt_scr[...] + cnt_inc

        @pl.when(i == GRID - 1)
        def _final():
            pooled = pooled_scr[...] / jnp.maximum(cnt_scr[...], 1.0)
            dn1 = (((1,), (1,)), ((), ()))
            z = jnp.maximum(
                lax.dot_general(pooled, l1w_ref[...], dn1,
                                preferred_element_type=jnp.float32)
                + l1b_ref[...], 0.0)
            o = (jnp.sum(z * l2w_ref[...], axis=1, keepdims=True)
                 + l2b_ref[0, 0])
            out_ref[...] = jax.nn.sigmoid(o)

    return pl.pallas_call(
        body,
        grid=(GRID,),
        in_specs=[
            pl.BlockSpec((BR, 2 * H), lambda i: (i, 0)),
            pl.BlockSpec((BR, H), lambda i: (i, 0)),
            pl.BlockSpec((BR, 1), lambda i: (i, 0)),
            pl.BlockSpec((1, H), lambda i: (0, 0)),
            pl.BlockSpec((BR, 1), lambda i: (i, 0)),
            pl.BlockSpec((32, H), lambda i: (0, 0)),
            pl.BlockSpec((1, 32), lambda i: (0, 0)),
            pl.BlockSpec((1, 32), lambda i: (0, 0)),
            pl.BlockSpec((1, 1), lambda i: (0, 0)),
        ],
        out_specs=pl.BlockSpec((G, 1), lambda i: (0, 0)),
        out_shape=jax.ShapeDtypeStruct((G, 1), jnp.float32),
        scratch_shapes=[
            pltpu.VMEM((G, G), jnp.float32),
            pltpu.VMEM((G, 1), jnp.float32),
        ],
    )(p, y3, dinv, g3b, batch2d, l1w, l1b, l2w, l2b)


def kernel(x, edge_index, batch, conv_w, conv_b, g1w, g1b, g2w, g2b,
           g3w, g3b, l1w, l1b, l2w, l2b):
    pad = EPAD - E
    src3d = jnp.concatenate(
        [edge_index[0],
         jnp.arange(pad, dtype=jnp.int32) % N]).reshape(NW, NCH, CH)
    dst3d = jnp.concatenate(
        [edge_index[1],
         N + (jnp.arange(pad, dtype=jnp.int32) % PADROWS)]
    ).reshape(NW, NCH, CH)
    cw = conv_w.reshape(1, 10)
    cb = conv_b.reshape(1, 1)

    dp = _sc_deg(dst3d)
    y1, dinv = _tc_pre(x, dp, cw, cb, g1w)
    p1 = _sc_agg(y1, src3d, dst3d)
    y2 = _tc_mid(p1, y1, dinv, g1b.reshape(1, H), g2w)
    p2 = _sc_agg(y2, src3d, dst3d)
    y3 = _tc_mid(p2, y2, dinv, g2b.reshape(1, H), g3w)
    p3 = _sc_agg(y3, src3d, dst3d)
    out = _tc_post(p3, y3, dinv, g3b.reshape(1, H),
                   batch.reshape(N, 1), l1w, l1b.reshape(1, 32),
                   l2w, l2b.reshape(1, 1))
    return out


# speedup vs baseline: 2.8886x; 1.0003x over previous
import functools

import jax
import jax.numpy as jnp
from jax import lax
from jax.experimental import pallas as pl
from jax.experimental.pallas import tpu as pltpu
from jax.experimental.pallas import tpu_sc as plsc

N = 10000
E = 320000
F = 128
H = 64
G = 64
NW = 32
CH = 128
NCH = 80
EPW = NCH * CH
EPAD = NW * EPW
PADROWS = 1024
AN = N + PADROWS
NBUF = 4
RPT = 1000
BR = 2000
GRID = N // BR


def _sc_deg(dst3d):
    mesh = plsc.VectorSubcoreMesh(core_axis_name="c", subcore_axis_name="s")

    @functools.partial(
        pl.kernel, mesh=mesh,
        compiler_params=pltpu.CompilerParams(use_tc_tiling_on_sc=False),
        out_type=jax.ShapeDtypeStruct((2 * N, 16), jnp.float32),
        scratch_types=[
            pltpu.VMEM((NCH, CH), jnp.int32),
            pltpu.VMEM((CH, 16), jnp.float32),
            pltpu.VMEM((RPT, 16), jnp.float32),
            pltpu.VMEM_SHARED((AN, 16), jnp.float32),
        ],
    )
    def k(dst_hbm, out_hbm, idx_v, ones_v, z_v, acc):
        c = lax.axis_index("c")
        s = lax.axis_index("s")
        wid = s * 2 + c

        def fill_ones(i, carry):
            ones_v[i, :] = jnp.ones((16,), jnp.float32)
            return carry
        lax.fori_loop(0, CH, fill_ones, 0)

        pltpu.sync_copy(dst_hbm.at[wid], idx_v)

        @pl.when(s < N // RPT)
        def _init():
            def fill_zero(i, carry):
                z_v[i, :] = jnp.zeros((16,), jnp.float32)
                return carry
            lax.fori_loop(0, RPT, fill_zero, 0)
            pltpu.sync_copy(z_v, acc.at[pl.ds(s * RPT, RPT)])

        plsc.subcore_barrier()

        def body(j, carry):
            pltpu.sync_copy(ones_v, acc.at[idx_v.at[j]], add=True)
            return carry
        lax.fori_loop(0, NCH, body, 0)

        plsc.subcore_barrier()

        @pl.when(s < N // RPT)
        def _writeback():
            pltpu.sync_copy(acc.at[pl.ds(s * RPT, RPT)],
                            out_hbm.at[pl.ds(c * N + s * RPT, RPT)])

    return k(dst3d)


def _sc_agg(y, src3d, dst3d):
    mesh = plsc.VectorSubcoreMesh(core_axis_name="c", subcore_axis_name="s")

    @functools.partial(
        pl.kernel, mesh=mesh,
        compiler_params=pltpu.CompilerParams(use_tc_tiling_on_sc=False),
        out_type=jax.ShapeDtypeStruct((N, 2 * H), jnp.float32),
        scratch_types=[
            pltpu.VMEM((NCH, CH), jnp.int32),
            pltpu.VMEM((NCH, CH), jnp.int32),
            [pltpu.VMEM((CH, H), jnp.float32) for _ in range(NBUF)],
            pltpu.VMEM_SHARED((AN, H), jnp.float32),
            pltpu.SemaphoreType.DMA,
            pltpu.SemaphoreType.DMA,
        ],
    )
    def k(y_hbm, src_hbm, dst_hbm, out_hbm, src_v, dst_v, bufs, acc,
          gsem, ssem):
        c = lax.axis_index("c")
        s = lax.axis_index("s")
        wid = s * 2 + c
        pltpu.sync_copy(src_hbm.at[wid], src_v)
        pltpu.sync_copy(dst_hbm.at[wid], dst_v)

        @pl.when(s < N // RPT)
        def _init():
            pltpu.sync_copy(y_hbm.at[pl.ds(s * RPT, RPT)],
                            acc.at[pl.ds(s * RPT, RPT)])

        plsc.subcore_barrier()

        for b in range(NBUF):
            pltpu.async_copy(y_hbm.at[src_v.at[b]], bufs[b], gsem)

        def body(i, carry):
            j0 = i * NBUF
            for b in range(NBUF):
                pltpu.make_async_copy(
                    y_hbm.at[src_v.at[j0 + b]], bufs[b], gsem).wait()
                pltpu.async_copy(bufs[b], acc.at[dst_v.at[j0 + b]], ssem,
                                 add=True)
            for b in range(NBUF):
                @pl.when(j0 + b + NBUF < NCH)
                def _refill(b=b):
                    pltpu.make_async_copy(
                        bufs[b], acc.at[dst_v.at[j0 + b]], ssem).wait()
                    pltpu.async_copy(
                        y_hbm.at[src_v.at[j0 + b + NBUF]], bufs[b], gsem)
            return carry
        lax.fori_loop(0, NCH // NBUF, body, 0)

        for b in range(NBUF):
            pltpu.make_async_copy(
                bufs[b], acc.at[dst_v.at[NCH - NBUF + b]], ssem).wait()

        plsc.subcore_barrier()

        @pl.when(s < N // RPT)
        def _writeback():
            pltpu.sync_copy(acc.at[pl.ds(s * RPT, RPT)],
                            out_hbm.at[pl.ds(s * RPT, RPT), pl.ds(c * H, H)])

    return k(y, src3d, dst3d)


def _conv_mat(wrow, lin, lout):
    ii = lax.broadcasted_iota(jnp.int32, (lin, lout), 0)
    jj = lax.broadcasted_iota(jnp.int32, (lin, lout), 1)
    t = ii - 2 * jj
    acc = jnp.zeros((lin, lout), jnp.float32)
    for tap in range(10):
        acc = acc + jnp.where(t == tap, wrow[:, tap:tap + 1], 0.0)
    return acc


def _tc_pre(x, dp, conv_w, conv_b, g1w):
    def body(x_ref, dp0_ref, dp1_ref, cw_ref, cb_ref, g1w_ref,
             y_ref, dinv_ref, w_scr, c1_scr):
        i = pl.program_id(0)

        @pl.when(i == 0)
        def _build():
            wrow = cw_ref[...]
            c1m = _conv_mat(wrow, F, 60)
            c2m = _conv_mat(wrow, 60, 26)
            c3m = _conv_mat(wrow, 26, 9)
            m = jnp.dot(jnp.dot(c1m, c2m, preferred_element_type=jnp.float32),
                        c3m, preferred_element_type=jnp.float32)
            w_scr[...] = jnp.dot(m, g1w_ref[...],
                                 preferred_element_type=jnp.float32)
            ssum = jnp.sum(wrow, keepdims=True)
            beta = cb_ref[...] * (ssum * ssum + ssum + 1.0)
            c1_scr[...] = beta * jnp.sum(g1w_ref[...], axis=0, keepdims=True)

        deg = dp0_ref[:, 0:1] + dp1_ref[:, 0:1] + 1.0
        dinv = lax.rsqrt(deg)
        dinv_ref[...] = dinv
        t0 = (jnp.dot(x_ref[...], w_scr[...],
                      preferred_element_type=jnp.float32) + c1_scr[...])
        y_ref[...] = dinv * t0

    return pl.pallas_call(
        body,
        grid=(GRID,),
        in_specs=[
            pl.BlockSpec((BR, F), lambda i: (i, 0)),
            pl.BlockSpec((BR, 16), lambda i: (i, 0)),
            pl.BlockSpec((BR, 16), lambda i: (i + GRID, 0)),
            pl.BlockSpec((1, 10), lambda i: (0, 0)),
            pl.BlockSpec((1, 1), lambda i: (0, 0)),
            pl.BlockSpec((9, H), lambda i: (0, 0)),
        ],
        out_specs=[
            pl.BlockSpec((BR, H), lambda i: (i, 0)),
            pl.BlockSpec((BR, 1), lambda i: (i, 0)),
        ],
        out_shape=[
            jax.ShapeDtypeStruct((N, H), jnp.float32),
            jax.ShapeDtypeStruct((N, 1), jnp.float32),
        ],
        scratch_shapes=[
            pltpu.VMEM((F, H), jnp.float32),
            pltpu.VMEM((1, H), jnp.float32),
        ],
    )(x, dp, dp, conv_w, conv_b, g1w)


def _tc_mid(p, y_prev, dinv, bias, w):
    def body(p_ref, yp_ref, dinv_ref, b_ref, w_ref, y_ref):
        dinv = dinv_ref[...]
        sagg = p_ref[:, :H] + p_ref[:, H:] - yp_ref[...]
        h = jnp.maximum(dinv * sagg + b_ref[...], 0.0)
        y_ref[...] = dinv * jnp.dot(h, w_ref[...],
                                    preferred_element_type=jnp.float32)

    return pl.pallas_call(
        body,
        grid=(GRID,),
        in_specs=[
            pl.BlockSpec((BR, 2 * H), lambda i: (i, 0)),
            pl.BlockSpec((BR, H), lambda i: (i, 0)),
            pl.BlockSpec((BR, 1), lambda i: (i, 0)),
            pl.BlockSpec((1, H), lambda i: (0, 0)),
            pl.BlockSpec((H, H), lambda i: (0, 0)),
        ],
        out_specs=pl.BlockSpec((BR, H), lambda i: (i, 0)),
        out_shape=jax.ShapeDtypeStruct((N, H), jnp.float32),
    )(p, y_prev, dinv, bias, w)


def _tc_post(p, y3, dinv, g3b, batch2d, l1w, l1b, l2w, l2b):
    def body(p_ref, y3_ref, dinv_ref, b_ref, bat_ref,
             l1w_ref, l1b_ref, l2w_ref, l2b_ref, out_ref,
             pooled_scr, cnt_scr):
        i = pl.program_id(0)
        h = (dinv_ref[...] * (p_ref[:, :H] + p_ref[:, H:] - y3_ref[...])
             + b_ref[...])
        gids = lax.broadcasted_iota(jnp.int32, (1, G), 1)
        onehot = (bat_ref[...] == gids).astype(jnp.float32)
        dn = (((0,), (0,)), ((), ()))
        pooled_inc = lax.dot_general(onehot, h, dn,
                                     preferred_element_type=jnp.float32)
        cnt_inc = lax.dot_general(onehot, jnp.ones((BR, 1), jnp.float32), dn,
                                  preferred_element_type=jnp.float32)

        @pl.when(i == 0)
        def _init():
            pooled_scr[...] = pooled_inc
            cnt_scr[...] = cnt_inc

        @pl.when(i > 0)
        def _acc():
            pooled_scr[...] = pooled_scr[...] + pooled_inc
            cnt_scr[...] = cnt_scr[...] + cnt_inc

        @pl.when(i == GRID - 1)
        def _final():
            pooled = pooled_scr[...] / jnp.maximum(cnt_scr[...], 1.0)
            dn1 = (((1,), (1,)), ((), ()))
            z = jnp.maximum(
                lax.dot_general(pooled, l1w_ref[...], dn1,
                                preferred_element_type=jnp.float32)
                + l1b_ref[...], 0.0)
            o = (jnp.sum(z * l2w_ref[...], axis=1, keepdims=True)
                 + l2b_ref[0, 0])
            out_ref[...] = jax.nn.sigmoid(o)

    return pl.pallas_call(
        body,
        grid=(GRID,),
        in_specs=[
            pl.BlockSpec((BR, 2 * H), lambda i: (i, 0)),
            pl.BlockSpec((BR, H), lambda i: (i, 0)),
            pl.BlockSpec((BR, 1), lambda i: (i, 0)),
            pl.BlockSpec((1, H), lambda i: (0, 0)),
            pl.BlockSpec((BR, 1), lambda i: (i, 0)),
            pl.BlockSpec((32, H), lambda i: (0, 0)),
            pl.BlockSpec((1, 32), lambda i: (0, 0)),
            pl.BlockSpec((1, 32), lambda i: (0, 0)),
            pl.BlockSpec((1, 1), lambda i: (0, 0)),
        ],
        out_specs=pl.BlockSpec((G, 1), lambda i: (0, 0)),
        out_shape=jax.ShapeDtypeStruct((G, 1), jnp.float32),
        scratch_shapes=[
            pltpu.VMEM((G, G), jnp.float32),
            pltpu.VMEM((G, 1), jnp.float32),
        ],
    )(p, y3, dinv, g3b, batch2d, l1w, l1b, l2w, l2b)


def kernel(x, edge_index, batch, conv_w, conv_b, g1w, g1b, g2w, g2b,
           g3w, g3b, l1w, l1b, l2w, l2b):
    pad = EPAD - E
    src3d = jnp.concatenate(
        [edge_index[0],
         jnp.arange(pad, dtype=jnp.int32) % N]).reshape(NW, NCH, CH)
    dst3d = jnp.concatenate(
        [edge_index[1],
         N + (jnp.arange(pad, dtype=jnp.int32) % PADROWS)]
    ).reshape(NW, NCH, CH)
    cw = conv_w.reshape(1, 10)
    cb = conv_b.reshape(1, 1)

    dp = _sc_deg(dst3d)
    y1, dinv = _tc_pre(x, dp, cw, cb, g1w)
    p1 = _sc_agg(y1, src3d, dst3d)
    y2 = _tc_mid(p1, y1, dinv, g1b.reshape(1, H), g2w)
    p2 = _sc_agg(y2, src3d, dst3d)
    y3 = _tc_mid(p2, y2, dinv, g2b.reshape(1, H), g3w)
    p3 = _sc_agg(y3, src3d, dst3d)
    out = _tc_post(p3, y3, dinv, g3b.reshape(1, H),
                   batch.reshape(N, 1), l1w, l1b.reshape(1, 32),
                   l2w, l2b.reshape(1, 1))
    return out


# speedup vs baseline: 2.9219x; 1.0115x over previous
import functools

import jax
import jax.numpy as jnp
from jax import lax
from jax.experimental import pallas as pl
from jax.experimental.pallas import tpu as pltpu
from jax.experimental.pallas import tpu_sc as plsc

N = 10000
E = 320000
F = 128
H = 64
G = 64
NW = 32
CH = 128
NCH = 80
EPW = NCH * CH
EPAD = NW * EPW
PADROWS = 1024
AN = N + PADROWS
NBUF = 5
RPT = 625
BR = 2000
GRID = N // BR


def _sc_deg(dst3d):
    mesh = plsc.VectorSubcoreMesh(core_axis_name="c", subcore_axis_name="s")

    @functools.partial(
        pl.kernel, mesh=mesh,
        compiler_params=pltpu.CompilerParams(use_tc_tiling_on_sc=False),
        out_type=jax.ShapeDtypeStruct((2 * N, 16), jnp.float32),
        scratch_types=[
            pltpu.VMEM((NCH, CH), jnp.int32),
            pltpu.VMEM((CH, 16), jnp.float32),
            pltpu.VMEM((RPT, 16), jnp.float32),
            pltpu.VMEM_SHARED((AN, 16), jnp.float32),
        ],
    )
    def k(dst_hbm, out_hbm, idx_v, ones_v, z_v, acc):
        c = lax.axis_index("c")
        s = lax.axis_index("s")
        wid = s * 2 + c

        def fill_ones(i, carry):
            ones_v[i, :] = jnp.ones((16,), jnp.float32)
            return carry
        lax.fori_loop(0, CH, fill_ones, 0)

        pltpu.sync_copy(dst_hbm.at[wid], idx_v)

        @pl.when(s < N // RPT)
        def _init():
            def fill_zero(i, carry):
                z_v[i, :] = jnp.zeros((16,), jnp.float32)
                return carry
            lax.fori_loop(0, RPT, fill_zero, 0)
            pltpu.sync_copy(z_v, acc.at[pl.ds(s * RPT, RPT)])

        plsc.subcore_barrier()

        def body(j, carry):
            pltpu.sync_copy(ones_v, acc.at[idx_v.at[j]], add=True)
            return carry
        lax.fori_loop(0, NCH, body, 0)

        plsc.subcore_barrier()

        @pl.when(s < N // RPT)
        def _writeback():
            pltpu.sync_copy(acc.at[pl.ds(s * RPT, RPT)],
                            out_hbm.at[pl.ds(c * N + s * RPT, RPT)])

    return k(dst3d)


def _sc_agg(y, src3d, dst3d):
    mesh = plsc.VectorSubcoreMesh(core_axis_name="c", subcore_axis_name="s")

    @functools.partial(
        pl.kernel, mesh=mesh,
        compiler_params=pltpu.CompilerParams(use_tc_tiling_on_sc=False),
        out_type=jax.ShapeDtypeStruct((N, 2 * H), jnp.float32),
        scratch_types=[
            pltpu.VMEM((NCH, CH), jnp.int32),
            pltpu.VMEM((NCH, CH), jnp.int32),
            [pltpu.VMEM((CH, H), jnp.float32) for _ in range(NBUF)],
            pltpu.VMEM_SHARED((AN, H), jnp.float32),
            pltpu.SemaphoreType.DMA,
            pltpu.SemaphoreType.DMA,
        ],
    )
    def k(y_hbm, src_hbm, dst_hbm, out_hbm, src_v, dst_v, bufs, acc,
          gsem, ssem):
        c = lax.axis_index("c")
        s = lax.axis_index("s")
        wid = s * 2 + c
        pltpu.sync_copy(src_hbm.at[wid], src_v)
        pltpu.sync_copy(dst_hbm.at[wid], dst_v)

        @pl.when(s < N // RPT)
        def _init():
            pltpu.sync_copy(y_hbm.at[pl.ds(s * RPT, RPT)],
                            acc.at[pl.ds(s * RPT, RPT)])

        plsc.subcore_barrier()

        for b in range(NBUF):
            pltpu.async_copy(y_hbm.at[src_v.at[b]], bufs[b], gsem)

        def body(i, carry):
            j0 = i * NBUF
            for b in range(NBUF):
                pltpu.make_async_copy(
                    y_hbm.at[src_v.at[j0 + b]], bufs[b], gsem).wait()
                pltpu.async_copy(bufs[b], acc.at[dst_v.at[j0 + b]], ssem,
                                 add=True)
            for b in range(NBUF):
                @pl.when(j0 + b + NBUF < NCH)
                def _refill(b=b):
                    pltpu.make_async_copy(
                        bufs[b], acc.at[dst_v.at[j0 + b]], ssem).wait()
                    pltpu.async_copy(
                        y_hbm.at[src_v.at[j0 + b + NBUF]], bufs[b], gsem)
            return carry
        lax.fori_loop(0, NCH // NBUF, body, 0)

        for b in range(NBUF):
            pltpu.make_async_copy(
                bufs[b], acc.at[dst_v.at[NCH - NBUF + b]], ssem).wait()

        plsc.subcore_barrier()

        @pl.when(s < N // RPT)
        def _writeback():
            pltpu.sync_copy(acc.at[pl.ds(s * RPT, RPT)],
                            out_hbm.at[pl.ds(s * RPT, RPT), pl.ds(c * H, H)])

    return k(y, src3d, dst3d)


def _conv_mat(wrow, lin, lout):
    ii = lax.broadcasted_iota(jnp.int32, (lin, lout), 0)
    jj = lax.broadcasted_iota(jnp.int32, (lin, lout), 1)
    t = ii - 2 * jj
    acc = jnp.zeros((lin, lout), jnp.float32)
    for tap in range(10):
        acc = acc + jnp.where(t == tap, wrow[:, tap:tap + 1], 0.0)
    return acc


def _tc_pre(x, dp, conv_w, conv_b, g1w):
    def body(x_ref, dp0_ref, dp1_ref, cw_ref, cb_ref, g1w_ref,
             y_ref, dinv_ref, w_scr, c1_scr):
        i = pl.program_id(0)

        @pl.when(i == 0)
        def _build():
            wrow = cw_ref[...]
            c1m = _conv_mat(wrow, F, 60)
            c2m = _conv_mat(wrow, 60, 26)
            c3m = _conv_mat(wrow, 26, 9)
            m = jnp.dot(jnp.dot(c1m, c2m, preferred_element_type=jnp.float32),
                        c3m, preferred_element_type=jnp.float32)
            w_scr[...] = jnp.dot(m, g1w_ref[...],
                                 preferred_element_type=jnp.float32)
            ssum = jnp.sum(wrow, keepdims=True)
            beta = cb_ref[...] * (ssum * ssum + ssum + 1.0)
            c1_scr[...] = beta * jnp.sum(g1w_ref[...], axis=0, keepdims=True)

        deg = dp0_ref[:, 0:1] + dp1_ref[:, 0:1] + 1.0
        dinv = lax.rsqrt(deg)
        dinv_ref[...] = dinv
        t0 = (jnp.dot(x_ref[...], w_scr[...],
                      preferred_element_type=jnp.float32) + c1_scr[...])
        y_ref[...] = dinv * t0

    return pl.pallas_call(
        body,
        grid=(GRID,),
        in_specs=[
            pl.BlockSpec((BR, F), lambda i: (i, 0)),
            pl.BlockSpec((BR, 16), lambda i: (i, 0)),
            pl.BlockSpec((BR, 16), lambda i: (i + GRID, 0)),
            pl.BlockSpec((1, 10), lambda i: (0, 0)),
            pl.BlockSpec((1, 1), lambda i: (0, 0)),
            pl.BlockSpec((9, H), lambda i: (0, 0)),
        ],
        out_specs=[
            pl.BlockSpec((BR, H), lambda i: (i, 0)),
            pl.BlockSpec((BR, 1), lambda i: (i, 0)),
        ],
        out_shape=[
            jax.ShapeDtypeStruct((N, H), jnp.float32),
            jax.ShapeDtypeStruct((N, 1), jnp.float32),
        ],
        scratch_shapes=[
            pltpu.VMEM((F, H), jnp.float32),
            pltpu.VMEM((1, H), jnp.float32),
        ],
    )(x, dp, dp, conv_w, conv_b, g1w)


def _tc_mid(p, y_prev, dinv, bias, w):
    def body(p_ref, yp_ref, dinv_ref, b_ref, w_ref, y_ref):
        dinv = dinv_ref[...]
        sagg = p_ref[:, :H] + p_ref[:, H:] - yp_ref[...]
        h = jnp.maximum(dinv * sagg + b_ref[...], 0.0)
        y_ref[...] = dinv * jnp.dot(h, w_ref[...],
                                    preferred_element_type=jnp.float32)

    return pl.pallas_call(
        body,
        grid=(GRID,),
        in_specs=[
            pl.BlockSpec((BR, 2 * H), lambda i: (i, 0)),
            pl.BlockSpec((BR, H), lambda i: (i, 0)),
            pl.BlockSpec((BR, 1), lambda i: (i, 0)),
            pl.BlockSpec((1, H), lambda i: (0, 0)),
            pl.BlockSpec((H, H), lambda i: (0, 0)),
        ],
        out_specs=pl.BlockSpec((BR, H), lambda i: (i, 0)),
        out_shape=jax.ShapeDtypeStruct((N, H), jnp.float32),
    )(p, y_prev, dinv, bias, w)


def _tc_post(p, y3, dinv, g3b, batch2d, l1w, l1b, l2w, l2b):
    def body(p_ref, y3_ref, dinv_ref, b_ref, bat_ref,
             l1w_ref, l1b_ref, l2w_ref, l2b_ref, out_ref,
             pooled_scr, cnt_scr):
        i = pl.program_id(0)
        h = (dinv_ref[...] * (p_ref[:, :H] + p_ref[:, H:] - y3_ref[...])
             + b_ref[...])
        gids = lax.broadcasted_iota(jnp.int32, (1, G), 1)
        onehot = (bat_ref[...] == gids).astype(jnp.float32)
        dn = (((0,), (0,)), ((), ()))
        pooled_inc = lax.dot_general(onehot, h, dn,
                                     preferred_element_type=jnp.float32)
        cnt_inc = lax.dot_general(onehot, jnp.ones((BR, 1), jnp.float32), dn,
                                  preferred_element_type=jnp.float32)

        @pl.when(i == 0)
        def _init():
            pooled_scr[...] = pooled_inc
            cnt_scr[...] = cnt_inc

        @pl.when(i > 0)
        def _acc():
            pooled_scr[...] = pooled_scr[...] + pooled_inc
            cnt_scr[...] = cnt_scr[...] + cnt_inc

        @pl.when(i == GRID - 1)
        def _final():
            pooled = pooled_scr[...] / jnp.maximum(cnt_scr[...], 1.0)
            dn1 = (((1,), (1,)), ((), ()))
            z = jnp.maximum(
                lax.dot_general(pooled, l1w_ref[...], dn1,
                                preferred_element_type=jnp.float32)
                + l1b_ref[...], 0.0)
            o = (jnp.sum(z * l2w_ref[...], axis=1, keepdims=True)
                 + l2b_ref[0, 0])
            out_ref[...] = jax.nn.sigmoid(o)

    return pl.pallas_call(
        body,
        grid=(GRID,),
        in_specs=[
            pl.BlockSpec((BR, 2 * H), lambda i: (i, 0)),
            pl.BlockSpec((BR, H), lambda i: (i, 0)),
            pl.BlockSpec((BR, 1), lambda i: (i, 0)),
            pl.BlockSpec((1, H), lambda i: (0, 0)),
            pl.BlockSpec((BR, 1), lambda i: (i, 0)),
            pl.BlockSpec((32, H), lambda i: (0, 0)),
            pl.BlockSpec((1, 32), lambda i: (0, 0)),
            pl.BlockSpec((1, 32), lambda i: (0, 0)),
            pl.BlockSpec((1, 1), lambda i: (0, 0)),
        ],
        out_specs=pl.BlockSpec((G, 1), lambda i: (0, 0)),
        out_shape=jax.ShapeDtypeStruct((G, 1), jnp.float32),
        scratch_shapes=[
            pltpu.VMEM((G, G), jnp.float32),
            pltpu.VMEM((G, 1), jnp.float32),
        ],
    )(p, y3, dinv, g3b, batch2d, l1w, l1b, l2w, l2b)


def kernel(x, edge_index, batch, conv_w, conv_b, g1w, g1b, g2w, g2b,
           g3w, g3b, l1w, l1b, l2w, l2b):
    pad = EPAD - E
    src3d = jnp.concatenate(
        [edge_index[0],
         jnp.arange(pad, dtype=jnp.int32) % N]).reshape(NW, NCH, CH)
    dst3d = jnp.concatenate(
        [edge_index[1],
         N + (jnp.arange(pad, dtype=jnp.int32) % PADROWS)]
    ).reshape(NW, NCH, CH)
    cw = conv_w.reshape(1, 10)
    cb = conv_b.reshape(1, 1)

    dp = _sc_deg(dst3d)
    y1, dinv = _tc_pre(x, dp, cw, cb, g1w)
    p1 = _sc_agg(y1, src3d, dst3d)
    y2 = _tc_mid(p1, y1, dinv, g1b.reshape(1, H), g2w)
    p2 = _sc_agg(y2, src3d, dst3d)
    y3 = _tc_mid(p2, y2, dinv, g2b.reshape(1, H), g3w)
    p3 = _sc_agg(y3, src3d, dst3d)
    out = _tc_post(p3, y3, dinv, g3b.reshape(1, H),
                   batch.reshape(N, 1), l1w, l1b.reshape(1, 32),
                   l2w, l2b.reshape(1, 1))
    return out
